# baseline (speedup 1.0000x reference)
import jax
import jax.numpy as jnp
from jax.experimental import pallas as pl

NA = 3
NO = 85


def _detect_kernel(x0a_ref, x0b_ref, x1_ref, x2_ref,
                   wt0_ref, wt1_ref, wt2_ref, b_ref,
                   out0_ref, out1_ref, out2_ref):
    xa = x0a_ref[0].astype(jnp.bfloat16)
    xb = x0b_ref[0].astype(jnp.bfloat16)
    ch = xa.shape[0]
    dn = (((0,), (0,)), ((), ()))
    for a in range(NA):
        za = jax.lax.dot_general(xa, wt0_ref[a, :ch], dn,
                                 preferred_element_type=jnp.float32)
        zb = jax.lax.dot_general(xb, wt0_ref[a, ch:], dn,
                                 preferred_element_type=jnp.float32)
        out0_ref[0, a] = za + zb + b_ref[0, a]
    for x_ref, wt_ref, lvl, out_ref in (
            (x1_ref, wt1_ref, 1, out1_ref),
            (x2_ref, wt2_ref, 2, out2_ref)):
        xv = x_ref[0].astype(jnp.bfloat16)
        for a in range(NA):
            z = jax.lax.dot_general(xv, wt_ref[a], dn,
                                    preferred_element_type=jnp.float32)
            out_ref[0, a] = z + b_ref[lvl, a]


def _pack_w(W, b):
    c = W.shape[1]
    wt = W.reshape(NA, NO, c).transpose(0, 2, 1).astype(jnp.bfloat16)
    br = b.reshape(NA, NO)
    return wt, br


@jax.jit
def _detect(x0, x1, x2, W0, b0, W1, b1, W2, b2):
    bs = x0.shape[0]
    shapes = [x.shape for x in (x0, x1, x2)]
    xr = [x.reshape(x.shape[0], x.shape[1], -1) for x in (x0, x1, x2)]
    packed = [_pack_w(W, b) for W, b in ((W0, b0), (W1, b1), (W2, b2))]
    wts = [p[0] for p in packed]
    brs = jnp.stack([p[1] for p in packed])
    c0h = shapes[0][1] // 2

    def x_spec(c, hw):
        return pl.BlockSpec((1, c, hw), lambda i: (i, 0, 0))

    def w_spec(c):
        return pl.BlockSpec((NA, c, NO), lambda i: (0, 0, 0))

    def o_spec(hw):
        return pl.BlockSpec((1, NA, hw, NO), lambda i: (i, 0, 0, 0))

    hw0 = shapes[0][2] * shapes[0][3]
    outs = pl.pallas_call(
        _detect_kernel,
        grid=(bs,),
        in_specs=(
            [pl.BlockSpec((1, c0h, hw0), lambda i: (i, 0, 0)),
             pl.BlockSpec((1, c0h, hw0), lambda i: (i, 1, 0))]
            + [x_spec(s[1], s[2] * s[3]) for s in shapes[1:]]
            + [w_spec(s[1]) for s in shapes]
            + [pl.BlockSpec((3, NA, NO), lambda i: (0, 0, 0))]
        ),
        out_specs=[o_spec(s[2] * s[3]) for s in shapes],
        out_shape=[
            jax.ShapeDtypeStruct((bs, NA, s[2] * s[3], NO), jnp.float32)
            for s in shapes],
    )(xr[0], xr[0], xr[1], xr[2], *wts, brs)
    return tuple(
        o.reshape(bs, NA, s[2], s[3], NO) for o, s in zip(outs, shapes))


def kernel(x0, x1, x2, W0, b0, W1, b1, W2, b2):
    return _detect(x0, x1, x2, W0, b0, W1, b1, W2, b2)
